# baseline (speedup 1.0000x reference)
import functools

import jax
import jax.numpy as jnp
from jax import lax
from jax.experimental import pallas as pl
from jax.experimental.pallas import tpu as pltpu
from jax.experimental.pallas import tpu_sc as plsc

_NC = 2
_NS = 16
_L = 16


def _spline_body(Q, B, K, D, T, b_per_w,
                 qt_hbm, knots_hbm, x0_hbm, x1_hbm, t_hbm, out_hbm,
                 t_v, q_v, a16_v, x0buf, x1buf, xt_v, o_v, idx_s,
                 isem0, isem1, osem0, osem1):
    isems = (isem0, isem1)
    osems = (osem0, osem1)
    wid = lax.axis_index("s") * _NC + lax.axis_index("c")
    b_base = wid * b_per_w

    pltpu.sync_copy(t_hbm, t_v)
    pltpu.sync_copy(qt_hbm, q_v)
    pltpu.sync_copy(x0_hbm.at[pl.ds(b_base, b_per_w)], x0buf)
    pltpu.sync_copy(x1_hbm.at[pl.ds(b_base, b_per_w)], x1buf)

    ones_i = jnp.full((_L,), 1, jnp.int32)
    zeros_i = jnp.full((_L,), 0, jnp.int32)
    tmax_i = jnp.full((_L,), T - 2, jnp.int32)
    eps_f = jnp.full((_L,), 1e-10, jnp.float32)

    @pl.loop(0, Q // _L)
    def _prelude(c):
        qv = q_v[pl.ds(c * _L, _L)]
        left = jnp.full((_L,), 0, jnp.int32)
        for tc in range(T // _L):
            tv = t_v[pl.ds(tc * _L, _L)]
            for l in range(_L):
                tb = jnp.broadcast_to(tv[l], (_L,))
                left = left + jnp.where(tb < qv, ones_i, zeros_i)
        idx = jnp.minimum(jnp.maximum(left - ones_i, zeros_i), tmax_i)
        idxp1 = idx + ones_i
        t0 = jnp.full((_L,), 0.0, jnp.float32)
        t1 = t0
        for tc in range(T // _L):
            tv = t_v[pl.ds(tc * _L, _L)]
            for l in range(_L):
                tb = jnp.broadcast_to(tv[l], (_L,))
                jv = jnp.full((_L,), tc * _L + l, jnp.int32)
                t0 = jnp.where(jv == idx, tb, t0)
                t1 = jnp.where(jv == idxp1, tb, t1)
        a = (qv - t0) / (t1 - t0 + eps_f)
        for l in range(_L):
            idx_s[c * _L + l] = idx[l]
            a16_v[c * _L + l] = jnp.broadcast_to(a[l], (_L,))

    def in_desc(slot, b):
        return pltpu.make_async_copy(
            knots_hbm.at[:, b, :],
            xt_v.at[pl.ds(slot * T + 1, K)], isems[slot])

    def out_desc(slot, b):
        return pltpu.make_async_copy(o_v.at[slot], out_hbm.at[b], osems[slot])

    def compute(slot, local_b):
        for c in range(D // _L):
            xt_v[slot * T, pl.ds(c * _L, _L)] = \
                x0buf[local_b, pl.ds(c * _L, _L)]
            xt_v[slot * T + T - 1, pl.ds(c * _L, _L)] = \
                x1buf[local_b, pl.ds(c * _L, _L)]

        nch = D // _L

        def row_off(qi):
            return idx_s[jnp.minimum(qi, Q - 1)] + slot * T

        def load_q(qi, r):
            va = a16_v[qi]
            v0s = [xt_v[r, pl.ds(c * _L, _L)] for c in range(nch)]
            v1s = [xt_v[r + 1, pl.ds(c * _L, _L)] for c in range(nch)]
            return tuple([va] + v0s + v1s)

        def store_q(qi, regs):
            va = regs[0]
            v0s = regs[1:1 + nch]
            v1s = regs[1 + nch:]
            for c in range(nch):
                o_v[slot, qi, pl.ds(c * _L, _L)] = \
                    v0s[c] + va * (v1s[c] - v0s[c])

        init = (row_off(1), load_q(0, row_off(0)))

        @plsc.parallel_loop(0, Q - 1, unroll=4, carry=init)
        def _per_q(qi, carry):
            ioff_nxt, regs = carry
            ioff_nxt2 = row_off(qi + 2)
            nxt = load_q(qi + 1, ioff_nxt)
            store_q(qi, regs)
            return (ioff_nxt2, nxt)

        store_q(Q - 1, _per_q[1])

    in_desc(0, b_base).start()
    in_desc(1, b_base + 1).start()

    @pl.loop(0, b_per_w, step=2)
    def _main(g):
        for slot in range(2):
            b = g + slot
            in_desc(slot, b_base).wait()

            @pl.when(b >= 2)
            def _():
                out_desc(slot, b_base).wait()

            compute(slot, b)
            out_desc(slot, b_base + b).start()

            @pl.when(b + 2 < b_per_w)
            def _():
                in_desc(slot, b_base + b + 2).start()

    out_desc(0, b_base).wait()
    out_desc(1, b_base).wait()


def kernel(query_t, knots, x0, x1, spline_discr):
    B, K, D = knots.shape
    T = K + 2
    Q = query_t.shape[0]
    n_workers = _NC * _NS
    b_per_w = B // n_workers

    t_lin = spline_discr[:, 0]
    x0r = x0[0]
    x1r = x1[0]
    knots_t = jnp.transpose(knots, (1, 0, 2))

    mesh = plsc.VectorSubcoreMesh(core_axis_name="c", subcore_axis_name="s")
    body = functools.partial(_spline_body, Q, B, K, D, T, b_per_w)
    sc_call = pl.kernel(
        body,
        out_type=jax.ShapeDtypeStruct((B, Q, D), jnp.float32),
        mesh=mesh,
        scratch_types=[
            pltpu.VMEM((T,), jnp.float32),
            pltpu.VMEM((Q,), jnp.float32),
            pltpu.VMEM((Q, _L), jnp.float32),
            pltpu.VMEM((b_per_w, D), jnp.float32),
            pltpu.VMEM((b_per_w, D), jnp.float32),
            pltpu.VMEM((2 * T, D), jnp.float32),
            pltpu.VMEM((2, Q, D), jnp.float32),
            pltpu.SMEM((Q,), jnp.int32),
            pltpu.SemaphoreType.DMA,
            pltpu.SemaphoreType.DMA,
            pltpu.SemaphoreType.DMA,
            pltpu.SemaphoreType.DMA,
        ],
    )
    return sc_call(query_t, knots_t, x0r, x1r, t_lin)


# speedup vs baseline: 1.2617x; 1.2617x over previous
import functools

import jax
import jax.numpy as jnp
from jax import lax
from jax.experimental import pallas as pl
from jax.experimental.pallas import tpu as pltpu
from jax.experimental.pallas import tpu_sc as plsc

_NC = 2
_NS = 16
_L = 16


def _spline_body(Q, B, K, D, T, b_per_w,
                 qt_hbm, knots_hbm, x0_hbm, x1_hbm, t_hbm, out_hbm,
                 t_v, q_v, a16_v, x0buf, x1buf, xt_v, o_v, idx_s,
                 isem0, isem1, osem0, osem1):
    isems = (isem0, isem1)
    osems = (osem0, osem1)
    wid = lax.axis_index("s") * _NC + lax.axis_index("c")
    b_base = wid * b_per_w

    pltpu.sync_copy(t_hbm, t_v)
    pltpu.sync_copy(qt_hbm, q_v)
    pltpu.sync_copy(x0_hbm.at[pl.ds(b_base, b_per_w)], x0buf)
    pltpu.sync_copy(x1_hbm.at[pl.ds(b_base, b_per_w)], x1buf)

    ones_i = jnp.full((_L,), 1, jnp.int32)
    zeros_i = jnp.full((_L,), 0, jnp.int32)
    tmax_i = jnp.full((_L,), T - 2, jnp.int32)
    eps_f = jnp.full((_L,), 1e-10, jnp.float32)

    @pl.loop(0, Q // _L)
    def _prelude(c):
        qv = q_v[pl.ds(c * _L, _L)]
        left = jnp.full((_L,), 0, jnp.int32)
        for tc in range(T // _L):
            tv = t_v[pl.ds(tc * _L, _L)]
            for l in range(_L):
                tb = jnp.broadcast_to(tv[l], (_L,))
                left = left + jnp.where(tb < qv, ones_i, zeros_i)
        idx = jnp.minimum(jnp.maximum(left - ones_i, zeros_i), tmax_i)
        idxp1 = idx + ones_i
        t0 = jnp.full((_L,), 0.0, jnp.float32)
        t1 = t0
        for tc in range(T // _L):
            tv = t_v[pl.ds(tc * _L, _L)]
            for l in range(_L):
                tb = jnp.broadcast_to(tv[l], (_L,))
                jv = jnp.full((_L,), tc * _L + l, jnp.int32)
                t0 = jnp.where(jv == idx, tb, t0)
                t1 = jnp.where(jv == idxp1, tb, t1)
        a = (qv - t0) / (t1 - t0 + eps_f)
        for l in range(_L):
            idx_s[c * _L + l] = idx[l]
            a16_v[c * _L + l] = jnp.broadcast_to(a[l], (_L,))

    def in_desc(slot, b):
        return pltpu.make_async_copy(
            knots_hbm.at[:, b, :],
            xt_v.at[pl.ds(slot * T + 1, K)], isems[slot])

    def out_desc(slot, b):
        return pltpu.make_async_copy(o_v.at[slot], out_hbm.at[b], osems[slot])

    def compute(slot, local_b):
        for c in range(D // _L):
            xt_v[slot * T, pl.ds(c * _L, _L)] = \
                x0buf[local_b, pl.ds(c * _L, _L)]
            xt_v[slot * T + T - 1, pl.ds(c * _L, _L)] = \
                x1buf[local_b, pl.ds(c * _L, _L)]

        nch = D // _L

        def row_off(qi):
            return idx_s[jnp.minimum(qi, Q - 1)] + slot * T

        def load_q(qi, r):
            va = a16_v[qi]
            v0s = [xt_v[r, pl.ds(c * _L, _L)] for c in range(nch)]
            v1s = [xt_v[r + 1, pl.ds(c * _L, _L)] for c in range(nch)]
            return tuple([va] + v0s + v1s)

        def store_q(qi, regs):
            va = regs[0]
            v0s = regs[1:1 + nch]
            v1s = regs[1 + nch:]
            for c in range(nch):
                o_v[slot, qi, pl.ds(c * _L, _L)] = \
                    v0s[c] + va * (v1s[c] - v0s[c])

        @plsc.parallel_loop(0, Q, unroll=2)
        def _per_q(qi):
            store_q(qi, load_q(qi, row_off(qi)))

    in_desc(0, b_base).start()
    in_desc(1, b_base + 1).start()

    @pl.loop(0, b_per_w, step=2)
    def _main(g):
        for slot in range(2):
            b = g + slot
            in_desc(slot, b_base).wait()

            @pl.when(b >= 2)
            def _():
                out_desc(slot, b_base).wait()

            compute(slot, b)
            out_desc(slot, b_base + b).start()

            @pl.when(b + 2 < b_per_w)
            def _():
                in_desc(slot, b_base + b + 2).start()

    out_desc(0, b_base).wait()
    out_desc(1, b_base).wait()


def kernel(query_t, knots, x0, x1, spline_discr):
    B, K, D = knots.shape
    T = K + 2
    Q = query_t.shape[0]
    n_workers = _NC * _NS
    b_per_w = B // n_workers

    t_lin = spline_discr[:, 0]
    x0r = x0[0]
    x1r = x1[0]
    knots_t = jnp.transpose(knots, (1, 0, 2))

    mesh = plsc.VectorSubcoreMesh(core_axis_name="c", subcore_axis_name="s")
    body = functools.partial(_spline_body, Q, B, K, D, T, b_per_w)
    sc_call = pl.kernel(
        body,
        out_type=jax.ShapeDtypeStruct((B, Q, D), jnp.float32),
        mesh=mesh,
        scratch_types=[
            pltpu.VMEM((T,), jnp.float32),
            pltpu.VMEM((Q,), jnp.float32),
            pltpu.VMEM((Q, _L), jnp.float32),
            pltpu.VMEM((b_per_w, D), jnp.float32),
            pltpu.VMEM((b_per_w, D), jnp.float32),
            pltpu.VMEM((2 * T, D), jnp.float32),
            pltpu.VMEM((2, Q, D), jnp.float32),
            pltpu.SMEM((Q,), jnp.int32),
            pltpu.SemaphoreType.DMA,
            pltpu.SemaphoreType.DMA,
            pltpu.SemaphoreType.DMA,
            pltpu.SemaphoreType.DMA,
        ],
    )
    return sc_call(query_t, knots_t, x0r, x1r, t_lin)
